# baseline (speedup 1.0000x reference)
import functools

import jax
import jax.numpy as jnp
from jax.experimental import pallas as pl
from jax.experimental.pallas import tpu as pltpu
from jax.experimental.shard_map import shard_map
from jax.sharding import Mesh, PartitionSpec as P


def _ceil_to(x, m):
    return -(-x // m) * m


def _conv_as_matmul3(w3, b3, t_in, seg):
    _, K, cin, cout = w3.shape
    t_out = t_in - K + 1
    v = jnp.concatenate(
        [w3, jnp.zeros((3, t_out, cin, cout), w3.dtype)], axis=1)
    flat = jnp.broadcast_to(
        v[:, None], (3, t_out, t_in + 1, cin, cout)).reshape(
            3, t_out * (t_in + 1), cin, cout)[:, :t_out * t_in]
    P = flat.reshape(3, t_out, t_in, cin, cout)
    W = P.transpose(2, 3, 0, 1, 4).reshape(t_in * cin, 3, t_out * cout)
    W = jnp.pad(W, ((0, 0), (0, 0), (0, seg - t_out * cout)))
    bias = jnp.broadcast_to(b3[:, None, :], (3, t_out, cout)).reshape(
        3, t_out * cout)
    bias = jnp.pad(bias, ((0, 0), (0, seg - t_out * cout)))
    return W.reshape(t_in * cin, 3 * seg), bias.reshape(1, 3 * seg)


def _gated(pre, seg):
    return jnp.maximum(
        pre[:, :seg] * jax.nn.sigmoid(pre[:, seg:2 * seg]) + pre[:, 2 * seg:],
        0.0)


def _fused_body(x_ref, a_ref, w1_ref, b1_ref, th_ref, w2_ref, b2_ref,
                o_ref, s1_ref, s2_ref, *, nb, n, m):
    seg1 = w1_ref.shape[1] // 3
    seg2 = w2_ref.shape[1] // 3
    rows = nb * n

    xb = x_ref[...].reshape(rows, x_ref.shape[2]).astype(jnp.bfloat16)
    h1 = _gated(
        jnp.dot(xb, w1_ref[...], preferred_element_type=jnp.float32)
        + b1_ref[...], seg1).astype(jnp.bfloat16)
    pj = jnp.dot(h1, th_ref[...],
                 preferred_element_type=jnp.float32).astype(jnp.bfloat16)
    adj = a_ref[...]
    mixed = jnp.concatenate(
        [jnp.dot(adj, pj[i * n:(i + 1) * n],
                 preferred_element_type=jnp.float32)
         for i in range(nb)], axis=0)
    h2 = jnp.maximum(mixed, 0.0).astype(jnp.bfloat16)
    out = _gated(
        jnp.dot(h2, w2_ref[...], preferred_element_type=jnp.float32)
        + b2_ref[...], seg2)[:, :m]
    o_ref[...] = out.reshape(nb, n, m).astype(jnp.bfloat16)
    s1_ref[...] = jnp.sum(out, axis=1, keepdims=True).reshape(nb, n, 1)
    s2_ref[...] = jnp.sum(out * out, axis=1, keepdims=True).reshape(nb, n, 1)


def _bn_body(t_ref, sc_ref, sh_ref, y_ref):
    y_ref[...] = (t_ref[...].astype(jnp.float32) * sc_ref[...][None, :, :]
                  + sh_ref[...][None, :, :])


def kernel(t1_w, t1_b, theta1, t2_w, t2_b, bn_gamma, bn_beta, X, A_hat):
    B, N, T, Cin = X.shape
    K = t1_w.shape[1]
    Cout = t1_w.shape[3]
    S = theta1.shape[1]
    T1 = T - (K - 1)
    T2 = T1 - (K - 1)
    M = T2 * Cout
    seg1 = _ceil_to(T1 * Cout, 128)
    seg2 = _ceil_to(M, 128)

    w1f, b1 = _conv_as_matmul3(t1_w, t1_b, T, seg1)
    w2f, b2 = _conv_as_matmul3(t2_w, t2_b, T1, seg2)
    w1 = w1f.astype(jnp.bfloat16)
    w2 = w2f.astype(jnp.bfloat16)
    eye = jnp.eye(T1, dtype=theta1.dtype)
    th = (eye[:, None, :, None] * theta1[None, :, None, :]).reshape(
        T1 * Cout, T1 * S)
    th = jnp.pad(th, ((0, seg1 - T1 * Cout), (0, 0))).astype(jnp.bfloat16)
    adj = A_hat.astype(jnp.bfloat16)
    x_lane = X.reshape(B, N, T * Cin)

    def _per_shard(x_sh, adj_, w1_, b1_, th_, w2_, b2_, g_, bt_, axis=None):
        bloc = x_sh.shape[0]
        nb = 8
        t3, s1, s2 = pl.pallas_call(
            functools.partial(_fused_body, nb=nb, n=N, m=M),
            out_shape=(jax.ShapeDtypeStruct((bloc, N, M), jnp.bfloat16),
                       jax.ShapeDtypeStruct((bloc, N, 1), jnp.float32),
                       jax.ShapeDtypeStruct((bloc, N, 1), jnp.float32)),
            grid=(bloc // nb,),
            in_specs=[
                pl.BlockSpec((nb, N, T * Cin), lambda i: (i, 0, 0)),
                pl.BlockSpec((N, N), lambda i: (0, 0)),
                pl.BlockSpec((T * Cin, 3 * seg1), lambda i: (0, 0)),
                pl.BlockSpec((1, 3 * seg1), lambda i: (0, 0)),
                pl.BlockSpec((seg1, T1 * S), lambda i: (0, 0)),
                pl.BlockSpec((T1 * S, 3 * seg2), lambda i: (0, 0)),
                pl.BlockSpec((1, 3 * seg2), lambda i: (0, 0)),
            ],
            out_specs=(
                pl.BlockSpec((nb, N, M), lambda i: (i, 0, 0)),
                pl.BlockSpec((nb, N, 1), lambda i: (i, 0, 0)),
                pl.BlockSpec((nb, N, 1), lambda i: (i, 0, 0)),
            ),
            compiler_params=pltpu.CompilerParams(
                dimension_semantics=("parallel",)),
        )(x_sh, adj_, w1_, b1_, th_, w2_, b2_)

        cnt = float(B * M)
        ssum = jnp.sum(s1[:, :, 0], axis=0)
        ssq = jnp.sum(s2[:, :, 0], axis=0)
        if axis is not None:
            ssum = jax.lax.psum(ssum, axis)
            ssq = jax.lax.psum(ssq, axis)
        mean = ssum / cnt
        var = jnp.maximum(ssq / cnt - mean * mean, 0.0)
        inv = jax.lax.rsqrt(var + 1e-5)
        g = g_[:, 0]
        scale = (g * inv)[:, None]
        shift = (bt_[:, 0] - mean * g * inv)[:, None]

        nb2 = 16 if bloc % 16 == 0 else nb
        return pl.pallas_call(
            _bn_body,
            out_shape=jax.ShapeDtypeStruct((bloc, N, M), jnp.float32),
            grid=(bloc // nb2,),
            in_specs=[
                pl.BlockSpec((nb2, N, M), lambda i: (i, 0, 0)),
                pl.BlockSpec((N, 1), lambda i: (0, 0)),
                pl.BlockSpec((N, 1), lambda i: (0, 0)),
            ],
            out_specs=pl.BlockSpec((nb2, N, M), lambda i: (i, 0, 0)),
            compiler_params=pltpu.CompilerParams(
                dimension_semantics=("parallel",)),
        )(t3, scale, shift)

    devs = jax.devices()
    ndev = 2 if (len(devs) >= 2 and B % 16 == 0) else 1
    if ndev > 1:
        mesh = Mesh(devs[:ndev], ("d",))
        rep = P()
        y = shard_map(
            functools.partial(_per_shard, axis="d"),
            mesh=mesh,
            in_specs=(P("d", None, None), rep, rep, rep, rep, rep, rep,
                      rep, rep),
            out_specs=P("d", None, None),
            check_rep=False,
        )(x_lane, adj, w1, b1, th, w2, b2, bn_gamma, bn_beta)
    else:
        y = _per_shard(x_lane, adj, w1, b1, th, w2, b2, bn_gamma, bn_beta)

    return y.reshape(B, N, T2, Cout)


# speedup vs baseline: 2.5391x; 2.5391x over previous
import functools

import jax
import jax.numpy as jnp
from jax.experimental import pallas as pl
from jax.experimental.pallas import tpu as pltpu


def _ceil_to(x, m):
    return -(-x // m) * m


def _conv_as_matmul3(w3, b3, t_in, seg):
    _, K, cin, cout = w3.shape
    t_out = t_in - K + 1
    v = jnp.concatenate(
        [w3, jnp.zeros((3, t_out, cin, cout), w3.dtype)], axis=1)
    flat = jnp.broadcast_to(
        v[:, None], (3, t_out, t_in + 1, cin, cout)).reshape(
            3, t_out * (t_in + 1), cin, cout)[:, :t_out * t_in]
    P = flat.reshape(3, t_out, t_in, cin, cout)
    W = P.transpose(2, 3, 0, 1, 4).reshape(t_in * cin, 3, t_out * cout)
    W = jnp.pad(W, ((0, 0), (0, 0), (0, seg - t_out * cout)))
    bias = jnp.broadcast_to(b3[:, None, :], (3, t_out, cout)).reshape(
        3, t_out * cout)
    bias = jnp.pad(bias, ((0, 0), (0, seg - t_out * cout)))
    return W.reshape(t_in * cin, 3 * seg), bias.reshape(1, 3 * seg)


def _gated(pre, seg):
    return jnp.maximum(
        pre[:, :seg] * jax.nn.sigmoid(pre[:, seg:2 * seg]) + pre[:, 2 * seg:],
        0.0)


def _fused_body(x_ref, a_ref, w1_ref, b1_ref, th_ref, w2_ref, b2_ref,
                o_ref, s1_ref, s2_ref, *, nb, n, m):
    seg1 = w1_ref.shape[1] // 3
    seg2 = w2_ref.shape[1] // 3
    rows = nb * n

    xb = x_ref[...].reshape(rows, x_ref.shape[2]).astype(jnp.bfloat16)
    h1 = _gated(
        jnp.dot(xb, w1_ref[...], preferred_element_type=jnp.float32)
        + b1_ref[...], seg1).astype(jnp.bfloat16)
    pj = jnp.dot(h1, th_ref[...],
                 preferred_element_type=jnp.float32).astype(jnp.bfloat16)
    adj = a_ref[...]
    mixed = jnp.concatenate(
        [jnp.dot(adj, pj[i * n:(i + 1) * n],
                 preferred_element_type=jnp.float32)
         for i in range(nb)], axis=0)
    h2 = jnp.maximum(mixed, 0.0).astype(jnp.bfloat16)
    out = _gated(
        jnp.dot(h2, w2_ref[...], preferred_element_type=jnp.float32)
        + b2_ref[...], seg2)[:, :m]
    o_ref[...] = out.reshape(nb, n, m).astype(jnp.bfloat16)
    s1_ref[...] = jnp.sum(out, axis=1, keepdims=True).reshape(nb, n, 1)
    s2_ref[...] = jnp.sum(out * out, axis=1, keepdims=True).reshape(nb, n, 1)


def _bn_body(t_ref, sc_ref, sh_ref, y_ref):
    y_ref[...] = (t_ref[...].astype(jnp.float32) * sc_ref[...][None, :, :]
                  + sh_ref[...][None, :, :])


def kernel(t1_w, t1_b, theta1, t2_w, t2_b, bn_gamma, bn_beta, X, A_hat):
    B, N, T, Cin = X.shape
    K = t1_w.shape[1]
    Cout = t1_w.shape[3]
    S = theta1.shape[1]
    T1 = T - (K - 1)
    T2 = T1 - (K - 1)
    M = T2 * Cout
    seg1 = _ceil_to(T1 * Cout, 128)
    seg2 = _ceil_to(M, 128)
    pjw = _ceil_to(T1 * S, 256)

    w1f, b1 = _conv_as_matmul3(t1_w, t1_b, T, seg1)
    w2f, b2 = _conv_as_matmul3(t2_w, t2_b, T1, seg2)
    w1 = w1f.astype(jnp.bfloat16)
    w2 = jnp.pad(w2f, ((0, pjw - T1 * S), (0, 0))).astype(jnp.bfloat16)
    eye = jnp.eye(T1, dtype=theta1.dtype)
    th = (eye[:, None, :, None] * theta1[None, :, None, :]).reshape(
        T1 * Cout, T1 * S)
    th = jnp.pad(th, ((0, seg1 - T1 * Cout), (0, pjw - T1 * S))
                 ).astype(jnp.bfloat16)
    adj = A_hat.astype(jnp.bfloat16)
    x_lane = X.reshape(B, N, T * Cin)

    nb = 8
    t3, s1, s2 = pl.pallas_call(
        functools.partial(_fused_body, nb=nb, n=N, m=M),
        out_shape=(jax.ShapeDtypeStruct((B, N, M), jnp.bfloat16),
                   jax.ShapeDtypeStruct((B, N, 1), jnp.float32),
                   jax.ShapeDtypeStruct((B, N, 1), jnp.float32)),
        grid=(B // nb,),
        in_specs=[
            pl.BlockSpec((nb, N, T * Cin), lambda i: (i, 0, 0)),
            pl.BlockSpec((N, N), lambda i: (0, 0)),
            pl.BlockSpec((T * Cin, 3 * seg1), lambda i: (0, 0)),
            pl.BlockSpec((1, 3 * seg1), lambda i: (0, 0)),
            pl.BlockSpec((seg1, pjw), lambda i: (0, 0)),
            pl.BlockSpec((pjw, 3 * seg2), lambda i: (0, 0)),
            pl.BlockSpec((1, 3 * seg2), lambda i: (0, 0)),
        ],
        out_specs=(
            pl.BlockSpec((nb, N, M), lambda i: (i, 0, 0)),
            pl.BlockSpec((nb, N, 1), lambda i: (i, 0, 0)),
            pl.BlockSpec((nb, N, 1), lambda i: (i, 0, 0)),
        ),
        compiler_params=pltpu.CompilerParams(
            dimension_semantics=("parallel",)),
    )(x_lane, adj, w1, b1, th, w2, b2)

    cnt = float(B * M)
    mean = jnp.sum(s1[:, :, 0], axis=0) / cnt
    var = jnp.maximum(jnp.sum(s2[:, :, 0], axis=0) / cnt - mean * mean, 0.0)
    inv = jax.lax.rsqrt(var + 1e-5)
    g = bn_gamma[:, 0]
    scale = (g * inv)[:, None]
    shift = (bn_beta[:, 0] - mean * g * inv)[:, None]

    nb2 = 16 if B % 16 == 0 else nb
    y = pl.pallas_call(
        _bn_body,
        out_shape=jax.ShapeDtypeStruct((B, N, M), jnp.float32),
        grid=(B // nb2,),
        in_specs=[
            pl.BlockSpec((nb2, N, M), lambda i: (i, 0, 0)),
            pl.BlockSpec((N, 1), lambda i: (0, 0)),
            pl.BlockSpec((N, 1), lambda i: (0, 0)),
        ],
        out_specs=pl.BlockSpec((nb2, N, M), lambda i: (i, 0, 0)),
        compiler_params=pltpu.CompilerParams(
            dimension_semantics=("parallel",)),
    )(t3, scale, shift)

    return y.reshape(B, N, T2, Cout)


# speedup vs baseline: 2.6035x; 1.0253x over previous
import functools

import jax
import jax.numpy as jnp
from jax.experimental import pallas as pl
from jax.experimental.pallas import tpu as pltpu


def _ceil_to(x, m):
    return -(-x // m) * m


def _conv_as_matmul3(w3, b3, t_in, seg):
    _, K, cin, cout = w3.shape
    t_out = t_in - K + 1
    v = jnp.concatenate(
        [w3, jnp.zeros((3, t_out, cin, cout), w3.dtype)], axis=1)
    flat = jnp.broadcast_to(
        v[:, None], (3, t_out, t_in + 1, cin, cout)).reshape(
            3, t_out * (t_in + 1), cin, cout)[:, :t_out * t_in]
    P = flat.reshape(3, t_out, t_in, cin, cout)
    W = P.transpose(2, 3, 0, 1, 4).reshape(t_in * cin, 3, t_out * cout)
    W = jnp.pad(W, ((0, 0), (0, 0), (0, seg - t_out * cout)))
    bias = jnp.broadcast_to(b3[:, None, :], (3, t_out, cout)).reshape(
        3, t_out * cout)
    bias = jnp.pad(bias, ((0, 0), (0, seg - t_out * cout)))
    return W.reshape(t_in * cin, 3 * seg), bias.reshape(1, 3 * seg)


def _gated(pre, seg):
    return jnp.maximum(
        pre[:, :seg] * jax.nn.sigmoid(pre[:, seg:2 * seg]) + pre[:, 2 * seg:],
        0.0)


def _fused_body(x_ref, a_ref, w1_ref, b1_ref, th_ref, w2_ref,
                o_ref, s1_ref, s2_ref, *, nb, n, m, one_lane):
    seg1 = w1_ref.shape[1] // 3
    seg2 = w2_ref.shape[1] // 3
    rows = nb * n

    xb = x_ref[...].reshape(rows, x_ref.shape[2]).astype(jnp.bfloat16)
    h1 = _gated(
        jnp.dot(xb, w1_ref[...], preferred_element_type=jnp.float32)
        + b1_ref[...], seg1).astype(jnp.bfloat16)
    pj = jnp.dot(h1, th_ref[...],
                 preferred_element_type=jnp.float32).astype(jnp.bfloat16)
    adj = a_ref[...]
    na = adj.shape[0]
    mixed = jnp.concatenate(
        [jnp.dot(adj, pj[i * na:(i + 1) * na],
                 preferred_element_type=jnp.float32)
         for i in range(rows // na)], axis=0)
    lane = jax.lax.broadcasted_iota(jnp.int32, (1, pj.shape[1]), 1)
    h2 = jnp.where(lane == one_lane, jnp.bfloat16(1.0),
                   jnp.maximum(mixed, 0.0).astype(jnp.bfloat16))
    out = _gated(
        jnp.dot(h2, w2_ref[...], preferred_element_type=jnp.float32),
        seg2)[:, :m]
    o_ref[...] = out.reshape(nb, n, m).astype(jnp.bfloat16)
    s1_ref[...] = jnp.sum(out, axis=1, keepdims=True).reshape(nb, n, 1)
    s2_ref[...] = jnp.sum(out * out, axis=1, keepdims=True).reshape(nb, n, 1)


def _bn_body(t_ref, sc_ref, sh_ref, y_ref):
    y_ref[...] = (t_ref[...].astype(jnp.float32) * sc_ref[...][None, :, :]
                  + sh_ref[...][None, :, :])


def kernel(t1_w, t1_b, theta1, t2_w, t2_b, bn_gamma, bn_beta, X, A_hat):
    B, N, T, Cin = X.shape
    K = t1_w.shape[1]
    Cout = t1_w.shape[3]
    S = theta1.shape[1]
    T1 = T - (K - 1)
    T2 = T1 - (K - 1)
    M = T2 * Cout
    seg1 = _ceil_to(T1 * Cout, 128)
    seg2 = _ceil_to(M, 128)
    pjw = _ceil_to(T1 * S, 256)

    w1f, b1 = _conv_as_matmul3(t1_w, t1_b, T, seg1)
    w2f, b2 = _conv_as_matmul3(t2_w, t2_b, T1, seg2)
    w1 = w1f.astype(jnp.bfloat16)
    w2 = jnp.concatenate(
        [w2f, b2, jnp.zeros((pjw - T1 * S - 1, 3 * seg2), w2f.dtype)],
        axis=0).astype(jnp.bfloat16)
    eye = jnp.eye(T1, dtype=theta1.dtype)
    th = (eye[:, None, :, None] * theta1[None, :, None, :]).reshape(
        T1 * Cout, T1 * S)
    th = jnp.pad(th, ((0, seg1 - T1 * Cout), (0, pjw - T1 * S))
                 ).astype(jnp.bfloat16)
    adj = jnp.kron(jnp.eye(2, dtype=A_hat.dtype), A_hat).astype(jnp.bfloat16)
    x_lane = X.reshape(B, N, T * Cin)

    nb = 16
    t3, s1, s2 = pl.pallas_call(
        functools.partial(_fused_body, nb=nb, n=N, m=M, one_lane=T1 * S),
        out_shape=(jax.ShapeDtypeStruct((B, N, M), jnp.bfloat16),
                   jax.ShapeDtypeStruct((B, N, 1), jnp.float32),
                   jax.ShapeDtypeStruct((B, N, 1), jnp.float32)),
        grid=(B // nb,),
        in_specs=[
            pl.BlockSpec((nb, N, T * Cin), lambda i: (i, 0, 0)),
            pl.BlockSpec((2 * N, 2 * N), lambda i: (0, 0)),
            pl.BlockSpec((T * Cin, 3 * seg1), lambda i: (0, 0)),
            pl.BlockSpec((1, 3 * seg1), lambda i: (0, 0)),
            pl.BlockSpec((seg1, pjw), lambda i: (0, 0)),
            pl.BlockSpec((pjw, 3 * seg2), lambda i: (0, 0)),
        ],
        out_specs=(
            pl.BlockSpec((nb, N, M), lambda i: (i, 0, 0)),
            pl.BlockSpec((nb, N, 1), lambda i: (i, 0, 0)),
            pl.BlockSpec((nb, N, 1), lambda i: (i, 0, 0)),
        ),
        compiler_params=pltpu.CompilerParams(
            dimension_semantics=("parallel",)),
    )(x_lane, adj, w1, b1, th, w2)

    cnt = float(B * M)
    mean = jnp.sum(s1[:, :, 0], axis=0) / cnt
    var = jnp.maximum(jnp.sum(s2[:, :, 0], axis=0) / cnt - mean * mean, 0.0)
    inv = jax.lax.rsqrt(var + 1e-5)
    g = bn_gamma[:, 0]
    scale = (g * inv)[:, None]
    shift = (bn_beta[:, 0] - mean * g * inv)[:, None]

    nb2 = 16 if B % 16 == 0 else nb
    y = pl.pallas_call(
        _bn_body,
        out_shape=jax.ShapeDtypeStruct((B, N, M), jnp.float32),
        grid=(B // nb2,),
        in_specs=[
            pl.BlockSpec((nb2, N, M), lambda i: (i, 0, 0)),
            pl.BlockSpec((N, 1), lambda i: (0, 0)),
            pl.BlockSpec((N, 1), lambda i: (0, 0)),
        ],
        out_specs=pl.BlockSpec((nb2, N, M), lambda i: (i, 0, 0)),
        compiler_params=pltpu.CompilerParams(
            dimension_semantics=("parallel",)),
    )(t3, scale, shift)

    return y.reshape(B, N, T2, Cout)


# speedup vs baseline: 2.8270x; 1.0859x over previous
import functools

import jax
import jax.numpy as jnp
from jax.experimental import pallas as pl
from jax.experimental.pallas import tpu as pltpu


def _ceil_to(x, m):
    return -(-x // m) * m


def _conv_as_matmul3(w3, b3, t_in, seg):
    _, K, cin, cout = w3.shape
    t_out = t_in - K + 1
    v = jnp.concatenate(
        [w3, jnp.zeros((3, t_out, cin, cout), w3.dtype)], axis=1)
    flat = jnp.broadcast_to(
        v[:, None], (3, t_out, t_in + 1, cin, cout)).reshape(
            3, t_out * (t_in + 1), cin, cout)[:, :t_out * t_in]
    P = flat.reshape(3, t_out, t_in, cin, cout)
    W = P.transpose(2, 3, 0, 1, 4).reshape(t_in * cin, 3, t_out * cout)
    W = jnp.pad(W, ((0, 0), (0, 0), (0, seg - t_out * cout)))
    bias = jnp.broadcast_to(b3[:, None, :], (3, t_out, cout)).reshape(
        3, t_out * cout)
    bias = jnp.pad(bias, ((0, 0), (0, seg - t_out * cout)))
    return W.reshape(t_in * cin, 3 * seg), bias.reshape(1, 3 * seg)


def _gated(pre, seg):
    return jnp.maximum(
        pre[:, :seg] * jax.nn.sigmoid(pre[:, seg:2 * seg]) + pre[:, 2 * seg:],
        0.0)


def _fused_body(x_ref, a_ref, w1_ref, b1_ref, th_ref, w2_ref,
                o_ref, *, nb, n, m, one_lane):
    seg1 = w1_ref.shape[1] // 3
    seg2 = w2_ref.shape[1] // 3
    rows = nb * n

    xb = x_ref[...].reshape(rows, x_ref.shape[2]).astype(jnp.bfloat16)
    h1 = _gated(
        jnp.dot(xb, w1_ref[...], preferred_element_type=jnp.float32)
        + b1_ref[...], seg1).astype(jnp.bfloat16)
    pj = jnp.dot(h1, th_ref[...],
                 preferred_element_type=jnp.float32).astype(jnp.bfloat16)
    adj = a_ref[...]
    na = adj.shape[0]
    mixed = jnp.concatenate(
        [jnp.dot(adj, pj[i * na:(i + 1) * na],
                 preferred_element_type=jnp.float32)
         for i in range(rows // na)], axis=0)
    lane = jax.lax.broadcasted_iota(jnp.int32, (1, pj.shape[1]), 1)
    h2 = jnp.where(lane == one_lane, jnp.bfloat16(1.0),
                   jnp.maximum(mixed, 0.0).astype(jnp.bfloat16))
    out = _gated(
        jnp.dot(h2, w2_ref[...], preferred_element_type=jnp.float32),
        seg2)[:, :m]
    o_ref[...] = out.reshape(nb, n, m).astype(jnp.bfloat16)


def _bn_body(t_ref, g_ref, b_ref, y_ref, *, cnt):
    x = t_ref[...].astype(jnp.float32)
    mean = jnp.sum(x, axis=(0, 2)) / cnt
    var = jnp.maximum(jnp.sum(x * x, axis=(0, 2)) / cnt - mean * mean, 0.0)
    inv = jax.lax.rsqrt(var + 1e-5)
    g = g_ref[...][:, 0]
    scale = (g * inv)[None, :, None]
    shift = (b_ref[...][:, 0] - mean * g * inv)[None, :, None]
    y_ref[...] = x * scale + shift


def kernel(t1_w, t1_b, theta1, t2_w, t2_b, bn_gamma, bn_beta, X, A_hat):
    B, N, T, Cin = X.shape
    K = t1_w.shape[1]
    Cout = t1_w.shape[3]
    S = theta1.shape[1]
    T1 = T - (K - 1)
    T2 = T1 - (K - 1)
    M = T2 * Cout
    seg1 = _ceil_to(T1 * Cout, 128)
    seg2 = _ceil_to(M, 128)
    pjw = _ceil_to(T1 * S, 256)

    w1f, b1 = _conv_as_matmul3(t1_w, t1_b, T, seg1)
    w2f, b2 = _conv_as_matmul3(t2_w, t2_b, T1, seg2)
    w1 = w1f.astype(jnp.bfloat16)
    w2 = jnp.concatenate(
        [w2f, b2, jnp.zeros((pjw - T1 * S - 1, 3 * seg2), w2f.dtype)],
        axis=0).astype(jnp.bfloat16)
    eye = jnp.eye(T1, dtype=theta1.dtype)
    th = (eye[:, None, :, None] * theta1[None, :, None, :]).reshape(
        T1 * Cout, T1 * S)
    th = jnp.pad(th, ((0, seg1 - T1 * Cout), (0, pjw - T1 * S))
                 ).astype(jnp.bfloat16)
    adj = jnp.kron(jnp.eye(2, dtype=A_hat.dtype), A_hat).astype(jnp.bfloat16)
    x_lane = X.reshape(B, N, T * Cin)

    nb = 16
    t3 = pl.pallas_call(
        functools.partial(_fused_body, nb=nb, n=N, m=M, one_lane=T1 * S),
        out_shape=jax.ShapeDtypeStruct((B, N, M), jnp.bfloat16),
        grid=(B // nb,),
        in_specs=[
            pl.BlockSpec((nb, N, T * Cin), lambda i: (i, 0, 0)),
            pl.BlockSpec((2 * N, 2 * N), lambda i: (0, 0)),
            pl.BlockSpec((T * Cin, 3 * seg1), lambda i: (0, 0)),
            pl.BlockSpec((1, 3 * seg1), lambda i: (0, 0)),
            pl.BlockSpec((seg1, pjw), lambda i: (0, 0)),
            pl.BlockSpec((pjw, 3 * seg2), lambda i: (0, 0)),
        ],
        out_specs=pl.BlockSpec((nb, N, M), lambda i: (i, 0, 0)),
        compiler_params=pltpu.CompilerParams(
            dimension_semantics=("parallel",)),
    )(x_lane, adj, w1, b1, th, w2)

    tn = 16 if N % 16 == 0 else 8
    y = pl.pallas_call(
        functools.partial(_bn_body, cnt=float(B * M)),
        out_shape=jax.ShapeDtypeStruct((B, N, M), jnp.float32),
        grid=(N // tn,),
        in_specs=[
            pl.BlockSpec((B, tn, M), lambda i: (0, i, 0)),
            pl.BlockSpec((tn, 1), lambda i: (i, 0)),
            pl.BlockSpec((tn, 1), lambda i: (i, 0)),
        ],
        out_specs=pl.BlockSpec((B, tn, M), lambda i: (0, i, 0)),
        compiler_params=pltpu.CompilerParams(
            dimension_semantics=("parallel",)),
    )(t3, bn_gamma, bn_beta)

    return y.reshape(B, N, T2, Cout)


# speedup vs baseline: 3.6879x; 1.3045x over previous
import functools

import jax
import jax.numpy as jnp
from jax.experimental import pallas as pl
from jax.experimental.pallas import tpu as pltpu


def _ceil_to(x, m):
    return -(-x // m) * m


def _conv_as_matmul3(w3, b3, t_in, seg):
    _, K, cin, cout = w3.shape
    t_out = t_in - K + 1
    v = jnp.concatenate(
        [w3, jnp.zeros((3, t_out, cin, cout), w3.dtype)], axis=1)
    flat = jnp.broadcast_to(
        v[:, None], (3, t_out, t_in + 1, cin, cout)).reshape(
            3, t_out * (t_in + 1), cin, cout)[:, :t_out * t_in]
    P = flat.reshape(3, t_out, t_in, cin, cout)
    W = P.transpose(2, 3, 0, 1, 4).reshape(t_in * cin, 3, t_out * cout)
    W = jnp.pad(W, ((0, 0), (0, 0), (0, seg - t_out * cout)))
    bias = jnp.broadcast_to(b3[:, None, :], (3, t_out, cout)).reshape(
        3, t_out * cout)
    bias = jnp.pad(bias, ((0, 0), (0, seg - t_out * cout)))
    return W.reshape(t_in * cin, 3 * seg), bias.reshape(3 * seg, 1)


def _gated_rows(pre, seg):
    return jnp.maximum(
        pre[:seg] * jax.nn.sigmoid(pre[seg:2 * seg]) + pre[2 * seg:],
        0.0)


def _fused_body(x_ref, a_ref, w1_ref, b1_ref, th_ref, w2_ref, one_ref,
                o_ref, s1_ref, s2_ref, *, nb, n, m):
    seg1 = w1_ref.shape[0] // 3
    seg2 = w2_ref.shape[0] // 3
    w1 = w1_ref[...]
    b1 = b1_ref[...]
    th = th_ref[...]
    w2 = w2_ref[...]
    akt = a_ref[...]
    onez = one_ref[...]
    s1_acc = jnp.zeros((1, 2 * n), jnp.float32)
    s2_acc = jnp.zeros((1, 2 * n), jnp.float32)

    for p in range(nb // 2):
        xp = jnp.concatenate(
            [x_ref[2 * p], x_ref[2 * p + 1]], axis=1).astype(jnp.bfloat16)
        pre1 = jnp.dot(w1, xp, preferred_element_type=jnp.float32) + b1
        h1 = _gated_rows(pre1, seg1).astype(jnp.bfloat16)
        pj = jnp.dot(th, h1, preferred_element_type=jnp.float32
                     ).astype(jnp.bfloat16)
        mixed = jnp.dot(pj, akt, preferred_element_type=jnp.float32)
        h2 = jnp.concatenate(
            [jnp.maximum(mixed, 0.0).astype(jnp.bfloat16), onez], axis=0)
        pre2 = jnp.dot(w2, h2, preferred_element_type=jnp.float32)
        out = _gated_rows(pre2, seg2)[:m]
        o_ref[2 * p] = out[:, :n].astype(jnp.bfloat16)
        o_ref[2 * p + 1] = out[:, n:].astype(jnp.bfloat16)
        s1_acc = s1_acc + jnp.sum(out, axis=0, keepdims=True)
        s2_acc = s2_acc + jnp.sum(out * out, axis=0, keepdims=True)

    s1_ref[...] = (s1_acc[:, :n] + s1_acc[:, n:]).reshape(1, 1, n)
    s2_ref[...] = (s2_acc[:, :n] + s2_acc[:, n:]).reshape(1, 1, n)


def _bn_body(t_ref, sc_ref, sh_ref, y_ref):
    y_ref[...] = (t_ref[...].astype(jnp.float32)
                  * sc_ref[...][None, :, :] + sh_ref[...][None, :, :])


def kernel(t1_w, t1_b, theta1, t2_w, t2_b, bn_gamma, bn_beta, X, A_hat):
    B, N, T, Cin = X.shape
    K = t1_w.shape[1]
    Cout = t1_w.shape[3]
    S = theta1.shape[1]
    T1 = T - (K - 1)
    T2 = T1 - (K - 1)
    M = T2 * Cout
    seg1 = _ceil_to(T1 * Cout, 128)
    seg2 = _ceil_to(M, 128)
    pjw = _ceil_to(T1 * S + 1, 8)

    w1f, b1 = _conv_as_matmul3(t1_w, t1_b, T, seg1)
    w2f, b2 = _conv_as_matmul3(t2_w, t2_b, T1, seg2)
    w1t = w1f.T.astype(jnp.bfloat16)
    w2t = jnp.concatenate(
        [w2f, b2.reshape(1, 3 * seg2),
         jnp.zeros((pjw - T1 * S - 1, 3 * seg2), w2f.dtype)],
        axis=0).T.astype(jnp.bfloat16)
    eye = jnp.eye(T1, dtype=theta1.dtype)
    tht = (eye[:, None, :, None] * theta1.T[None, :, None, :]).reshape(
        T1 * S, T1 * Cout)
    tht = jnp.pad(tht, ((0, 0), (0, seg1 - T1 * Cout))).astype(jnp.bfloat16)
    akt = jnp.kron(jnp.eye(2, dtype=A_hat.dtype), A_hat.T
                   ).astype(jnp.bfloat16)
    row = jax.lax.broadcasted_iota(jnp.int32, (pjw - T1 * S, 1), 0)
    onez = jnp.where(row == 0, jnp.bfloat16(1.0), jnp.bfloat16(0.0))
    onez = jnp.broadcast_to(onez, (pjw - T1 * S, 2 * N))

    x_t = X.transpose(0, 2, 3, 1).reshape(B, T * Cin, N)

    nb = 8
    nsteps = B // nb
    t3, s1, s2 = pl.pallas_call(
        functools.partial(_fused_body, nb=nb, n=N, m=M),
        out_shape=(jax.ShapeDtypeStruct((B, M, N), jnp.bfloat16),
                   jax.ShapeDtypeStruct((nsteps, 1, N), jnp.float32),
                   jax.ShapeDtypeStruct((nsteps, 1, N), jnp.float32)),
        grid=(nsteps,),
        in_specs=[
            pl.BlockSpec((nb, T * Cin, N), lambda i: (i, 0, 0)),
            pl.BlockSpec((2 * N, 2 * N), lambda i: (0, 0)),
            pl.BlockSpec((3 * seg1, T * Cin), lambda i: (0, 0)),
            pl.BlockSpec((3 * seg1, 1), lambda i: (0, 0)),
            pl.BlockSpec((T1 * S, seg1), lambda i: (0, 0)),
            pl.BlockSpec((3 * seg2, pjw), lambda i: (0, 0)),
            pl.BlockSpec((pjw - T1 * S, 2 * N), lambda i: (0, 0)),
        ],
        out_specs=(
            pl.BlockSpec((nb, M, N), lambda i: (i, 0, 0)),
            pl.BlockSpec((1, 1, N), lambda i: (i, 0, 0)),
            pl.BlockSpec((1, 1, N), lambda i: (i, 0, 0)),
        ),
        compiler_params=pltpu.CompilerParams(
            dimension_semantics=("parallel",)),
    )(x_t, akt, w1t, b1, tht, w2t, onez)

    cnt = float(B * M)
    mean = jnp.sum(s1[:, 0, :], axis=0) / cnt
    var = jnp.maximum(jnp.sum(s2[:, 0, :], axis=0) / cnt - mean * mean, 0.0)
    inv = jax.lax.rsqrt(var + 1e-5)
    g = bn_gamma[:, 0]
    scale = (g * inv)[None, :]
    shift = (bn_beta[:, 0] - mean * g * inv)[None, :]

    nb2 = 32 if B % 32 == 0 else nb
    y = pl.pallas_call(
        _bn_body,
        out_shape=jax.ShapeDtypeStruct((B, M, N), jnp.float32),
        grid=(B // nb2,),
        in_specs=[
            pl.BlockSpec((nb2, M, N), lambda i: (i, 0, 0)),
            pl.BlockSpec((1, N), lambda i: (0, 0)),
            pl.BlockSpec((1, N), lambda i: (0, 0)),
        ],
        out_specs=pl.BlockSpec((nb2, M, N), lambda i: (i, 0, 0)),
        compiler_params=pltpu.CompilerParams(
            dimension_semantics=("parallel",)),
    )(t3, scale, shift)

    return y.reshape(B, T2, Cout, N).transpose(0, 3, 1, 2)


# speedup vs baseline: 3.7208x; 1.0089x over previous
import functools

import jax
import jax.numpy as jnp
from jax.experimental import pallas as pl
from jax.experimental.pallas import tpu as pltpu


def _ceil_to(x, m):
    return -(-x // m) * m


def _conv_as_matmul3(w3, b3, t_in, seg):
    _, K, cin, cout = w3.shape
    t_out = t_in - K + 1
    v = jnp.concatenate(
        [w3, jnp.zeros((3, t_out, cin, cout), w3.dtype)], axis=1)
    flat = jnp.broadcast_to(
        v[:, None], (3, t_out, t_in + 1, cin, cout)).reshape(
            3, t_out * (t_in + 1), cin, cout)[:, :t_out * t_in]
    P = flat.reshape(3, t_out, t_in, cin, cout)
    W = P.transpose(2, 3, 0, 1, 4).reshape(t_in * cin, 3, t_out * cout)
    W = jnp.pad(W, ((0, 0), (0, 0), (0, seg - t_out * cout)))
    bias = jnp.broadcast_to(b3[:, None, :], (3, t_out, cout)).reshape(
        3, t_out * cout)
    bias = jnp.pad(bias, ((0, 0), (0, seg - t_out * cout)))
    return W.reshape(t_in * cin, 3 * seg), bias.reshape(3 * seg, 1)


def _gated_rows(pre, seg):
    return jnp.maximum(
        pre[:seg] * jax.nn.sigmoid(pre[seg:2 * seg]) + pre[2 * seg:],
        0.0)


def _fused_body(x_ref, a_ref, w1_ref, b1_ref, th_ref, w2_ref, one_ref,
                o_ref, s1_ref, s2_ref, *, nb, n, m):
    seg1 = w1_ref.shape[0] // 3
    seg2 = w2_ref.shape[0] // 3
    w1 = w1_ref[...]
    b1 = b1_ref[...]
    th = th_ref[...]
    w2 = w2_ref[...]
    akt = a_ref[...]
    onez = one_ref[...]
    s1_acc = jnp.zeros((1, 2 * n), jnp.float32)
    s2_acc = jnp.zeros((1, 2 * n), jnp.float32)

    for p in range(nb // 2):
        xp = jnp.concatenate(
            [x_ref[2 * p], x_ref[2 * p + 1]], axis=1).astype(jnp.bfloat16)
        pre1 = jnp.dot(w1, xp, preferred_element_type=jnp.float32) + b1
        h1 = _gated_rows(pre1, seg1).astype(jnp.bfloat16)
        pj = jnp.dot(th, h1, preferred_element_type=jnp.float32
                     ).astype(jnp.bfloat16)
        mixed = jnp.dot(pj, akt, preferred_element_type=jnp.float32)
        h2 = jnp.concatenate(
            [jnp.maximum(mixed, 0.0).astype(jnp.bfloat16), onez], axis=0)
        pre2 = jnp.dot(w2, h2, preferred_element_type=jnp.float32)
        out = _gated_rows(pre2, seg2)[:m]
        o_ref[2 * p] = out[:, :n].astype(jnp.bfloat16)
        o_ref[2 * p + 1] = out[:, n:].astype(jnp.bfloat16)
        s1_acc = s1_acc + jnp.sum(out, axis=0, keepdims=True)
        s2_acc = s2_acc + jnp.sum(out * out, axis=0, keepdims=True)

    s1_ref[...] = (s1_acc[:, :n] + s1_acc[:, n:]).reshape(1, 1, n)
    s2_ref[...] = (s2_acc[:, :n] + s2_acc[:, n:]).reshape(1, 1, n)


def _bn_body(t_ref, sc_ref, sh_ref, y_ref):
    y_ref[...] = (t_ref[...].astype(jnp.float32)
                  * sc_ref[...][None, :, :] + sh_ref[...][None, :, :])


def kernel(t1_w, t1_b, theta1, t2_w, t2_b, bn_gamma, bn_beta, X, A_hat):
    B, N, T, Cin = X.shape
    K = t1_w.shape[1]
    Cout = t1_w.shape[3]
    S = theta1.shape[1]
    T1 = T - (K - 1)
    T2 = T1 - (K - 1)
    M = T2 * Cout
    seg1 = _ceil_to(T1 * Cout, 128)
    seg2 = _ceil_to(M, 128)
    pjw = _ceil_to(T1 * S + 1, 8)

    w1f, b1 = _conv_as_matmul3(t1_w, t1_b, T, seg1)
    w2f, b2 = _conv_as_matmul3(t2_w, t2_b, T1, seg2)
    w1t = w1f.T.astype(jnp.bfloat16)
    w2t = jnp.concatenate(
        [w2f, b2.reshape(1, 3 * seg2),
         jnp.zeros((pjw - T1 * S - 1, 3 * seg2), w2f.dtype)],
        axis=0).T.astype(jnp.bfloat16)
    eye = jnp.eye(T1, dtype=theta1.dtype)
    tht = (eye[:, None, :, None] * theta1.T[None, :, None, :]).reshape(
        T1 * S, T1 * Cout)
    tht = jnp.pad(tht, ((0, 0), (0, seg1 - T1 * Cout))).astype(jnp.bfloat16)
    akt = jnp.kron(jnp.eye(2, dtype=A_hat.dtype), A_hat.T
                   ).astype(jnp.bfloat16)
    row = jax.lax.broadcasted_iota(jnp.int32, (pjw - T1 * S, 1), 0)
    onez = jnp.where(row == 0, jnp.bfloat16(1.0), jnp.bfloat16(0.0))
    onez = jnp.broadcast_to(onez, (pjw - T1 * S, 2 * N))

    x_t = X.transpose(0, 2, 3, 1).reshape(B, T * Cin, N)

    nb = 16
    nsteps = B // nb
    t3, s1, s2 = pl.pallas_call(
        functools.partial(_fused_body, nb=nb, n=N, m=M),
        out_shape=(jax.ShapeDtypeStruct((B, M, N), jnp.bfloat16),
                   jax.ShapeDtypeStruct((nsteps, 1, N), jnp.float32),
                   jax.ShapeDtypeStruct((nsteps, 1, N), jnp.float32)),
        grid=(nsteps,),
        in_specs=[
            pl.BlockSpec((nb, T * Cin, N), lambda i: (i, 0, 0)),
            pl.BlockSpec((2 * N, 2 * N), lambda i: (0, 0)),
            pl.BlockSpec((3 * seg1, T * Cin), lambda i: (0, 0)),
            pl.BlockSpec((3 * seg1, 1), lambda i: (0, 0)),
            pl.BlockSpec((T1 * S, seg1), lambda i: (0, 0)),
            pl.BlockSpec((3 * seg2, pjw), lambda i: (0, 0)),
            pl.BlockSpec((pjw - T1 * S, 2 * N), lambda i: (0, 0)),
        ],
        out_specs=(
            pl.BlockSpec((nb, M, N), lambda i: (i, 0, 0)),
            pl.BlockSpec((1, 1, N), lambda i: (i, 0, 0)),
            pl.BlockSpec((1, 1, N), lambda i: (i, 0, 0)),
        ),
        compiler_params=pltpu.CompilerParams(
            dimension_semantics=("parallel",)),
    )(x_t, akt, w1t, b1, tht, w2t, onez)

    cnt = float(B * M)
    mean = jnp.sum(s1[:, 0, :], axis=0) / cnt
    var = jnp.maximum(jnp.sum(s2[:, 0, :], axis=0) / cnt - mean * mean, 0.0)
    inv = jax.lax.rsqrt(var + 1e-5)
    g = bn_gamma[:, 0]
    scale = (g * inv)[None, :]
    shift = (bn_beta[:, 0] - mean * g * inv)[None, :]

    nb2 = 32 if B % 32 == 0 else nb
    y = pl.pallas_call(
        _bn_body,
        out_shape=jax.ShapeDtypeStruct((B, M, N), jnp.float32),
        grid=(B // nb2,),
        in_specs=[
            pl.BlockSpec((nb2, M, N), lambda i: (i, 0, 0)),
            pl.BlockSpec((1, N), lambda i: (0, 0)),
            pl.BlockSpec((1, N), lambda i: (0, 0)),
        ],
        out_specs=pl.BlockSpec((nb2, M, N), lambda i: (i, 0, 0)),
        compiler_params=pltpu.CompilerParams(
            dimension_semantics=("parallel",)),
    )(t3, scale, shift)

    return y.reshape(B, T2, Cout, N).transpose(0, 3, 1, 2)


# speedup vs baseline: 3.8104x; 1.0241x over previous
import functools

import jax
import jax.numpy as jnp
from jax.experimental import pallas as pl
from jax.experimental.pallas import tpu as pltpu


def _ceil_to(x, m):
    return -(-x // m) * m


def _conv_as_matmul3(w3, b3, t_in, seg):
    _, K, cin, cout = w3.shape
    t_out = t_in - K + 1
    v = jnp.concatenate(
        [w3, jnp.zeros((3, t_out, cin, cout), w3.dtype)], axis=1)
    flat = jnp.broadcast_to(
        v[:, None], (3, t_out, t_in + 1, cin, cout)).reshape(
            3, t_out * (t_in + 1), cin, cout)[:, :t_out * t_in]
    P = flat.reshape(3, t_out, t_in, cin, cout)
    W = P.transpose(2, 3, 0, 1, 4).reshape(t_in * cin, 3, t_out * cout)
    W = jnp.pad(W, ((0, 0), (0, 0), (0, seg - t_out * cout)))
    bias = jnp.broadcast_to(b3[:, None, :], (3, t_out, cout)).reshape(
        3, t_out * cout)
    bias = jnp.pad(bias, ((0, 0), (0, seg - t_out * cout)))
    return W.reshape(t_in * cin, 3 * seg), bias.reshape(3 * seg, 1)


def _gated_rows(pre, seg):
    return jnp.maximum(
        pre[:seg] * jax.nn.sigmoid(pre[seg:2 * seg]) + pre[2 * seg:],
        0.0)


def _fused_body(x_ref, a_ref, w1_ref, b1_ref, th_ref, w2_ref, one_ref,
                o_ref, s1_ref, s2_ref, *, nb, n, m):
    seg1 = w1_ref.shape[0] // 3
    seg2 = w2_ref.shape[0] // 3
    w1 = w1_ref[...]
    b1 = b1_ref[...]
    th = th_ref[...]
    w2 = w2_ref[...]
    akt = a_ref[...]
    onez = one_ref[...]
    s1_acc = jnp.zeros((1, 2 * n), jnp.float32)
    s2_acc = jnp.zeros((1, 2 * n), jnp.float32)

    for p in range(nb // 2):
        xp = jnp.concatenate(
            [x_ref[2 * p], x_ref[2 * p + 1]], axis=1).astype(jnp.bfloat16)
        pre1 = jnp.dot(w1, xp, preferred_element_type=jnp.float32) + b1
        h1 = _gated_rows(pre1, seg1).astype(jnp.bfloat16)
        pj = jnp.dot(th, h1, preferred_element_type=jnp.float32
                     ).astype(jnp.bfloat16)
        mixed = jnp.dot(pj, akt, preferred_element_type=jnp.float32)
        h2 = jnp.concatenate(
            [jnp.maximum(mixed, 0.0).astype(jnp.bfloat16), onez], axis=0)
        pre2 = jnp.dot(w2, h2, preferred_element_type=jnp.float32)
        out = _gated_rows(pre2, seg2)[:m]
        o_ref[2 * p] = out[:, :n].astype(jnp.bfloat16)
        o_ref[2 * p + 1] = out[:, n:].astype(jnp.bfloat16)
        s1_acc = s1_acc + jnp.sum(out, axis=0, keepdims=True)
        s2_acc = s2_acc + jnp.sum(out * out, axis=0, keepdims=True)

    s1_ref[...] = (s1_acc[:, :n] + s1_acc[:, n:]).reshape(1, 1, n)
    s2_ref[...] = (s2_acc[:, :n] + s2_acc[:, n:]).reshape(1, 1, n)


def _bn_body(t_ref, sc_ref, sh_ref, y_ref):
    y_ref[...] = (t_ref[...].astype(jnp.float32)
                  * sc_ref[...][None, :, :] + sh_ref[...][None, :, :])


def kernel(t1_w, t1_b, theta1, t2_w, t2_b, bn_gamma, bn_beta, X, A_hat):
    B, N, T, Cin = X.shape
    K = t1_w.shape[1]
    Cout = t1_w.shape[3]
    S = theta1.shape[1]
    T1 = T - (K - 1)
    T2 = T1 - (K - 1)
    M = T2 * Cout
    seg1 = _ceil_to(T1 * Cout, 128)
    seg2 = _ceil_to(M, 128)
    pjw = _ceil_to(T1 * S + 1, 8)

    w1f, b1 = _conv_as_matmul3(t1_w, t1_b, T, seg1)
    w2f, b2 = _conv_as_matmul3(t2_w, t2_b, T1, seg2)
    w1t = w1f.T.astype(jnp.bfloat16)
    w2t = jnp.concatenate(
        [w2f, b2.reshape(1, 3 * seg2),
         jnp.zeros((pjw - T1 * S - 1, 3 * seg2), w2f.dtype)],
        axis=0).T.astype(jnp.bfloat16)
    eye = jnp.eye(T1, dtype=theta1.dtype)
    tht = (eye[:, None, :, None] * theta1.T[None, :, None, :]).reshape(
        T1 * S, T1 * Cout)
    tht = jnp.pad(tht, ((0, 0), (0, seg1 - T1 * Cout))).astype(jnp.bfloat16)
    akt = jnp.kron(jnp.eye(2, dtype=A_hat.dtype), A_hat.T
                   ).astype(jnp.bfloat16)
    row = jax.lax.broadcasted_iota(jnp.int32, (pjw - T1 * S, 1), 0)
    onez = jnp.where(row == 0, jnp.bfloat16(1.0), jnp.bfloat16(0.0))
    onez = jnp.broadcast_to(onez, (pjw - T1 * S, 2 * N))

    x_t = X.transpose(0, 2, 3, 1).reshape(B, T * Cin, N)

    nb = 32
    nsteps = B // nb
    t3, s1, s2 = pl.pallas_call(
        functools.partial(_fused_body, nb=nb, n=N, m=M),
        out_shape=(jax.ShapeDtypeStruct((B, M, N), jnp.bfloat16),
                   jax.ShapeDtypeStruct((nsteps, 1, N), jnp.float32),
                   jax.ShapeDtypeStruct((nsteps, 1, N), jnp.float32)),
        grid=(nsteps,),
        in_specs=[
            pl.BlockSpec((nb, T * Cin, N), lambda i: (i, 0, 0)),
            pl.BlockSpec((2 * N, 2 * N), lambda i: (0, 0)),
            pl.BlockSpec((3 * seg1, T * Cin), lambda i: (0, 0)),
            pl.BlockSpec((3 * seg1, 1), lambda i: (0, 0)),
            pl.BlockSpec((T1 * S, seg1), lambda i: (0, 0)),
            pl.BlockSpec((3 * seg2, pjw), lambda i: (0, 0)),
            pl.BlockSpec((pjw - T1 * S, 2 * N), lambda i: (0, 0)),
        ],
        out_specs=(
            pl.BlockSpec((nb, M, N), lambda i: (i, 0, 0)),
            pl.BlockSpec((1, 1, N), lambda i: (i, 0, 0)),
            pl.BlockSpec((1, 1, N), lambda i: (i, 0, 0)),
        ),
        compiler_params=pltpu.CompilerParams(
            dimension_semantics=("parallel",)),
    )(x_t, akt, w1t, b1, tht, w2t, onez)

    cnt = float(B * M)
    mean = jnp.sum(s1[:, 0, :], axis=0) / cnt
    var = jnp.maximum(jnp.sum(s2[:, 0, :], axis=0) / cnt - mean * mean, 0.0)
    inv = jax.lax.rsqrt(var + 1e-5)
    g = bn_gamma[:, 0]
    scale = (g * inv)[None, :]
    shift = (bn_beta[:, 0] - mean * g * inv)[None, :]

    nb2 = 32 if B % 32 == 0 else nb
    y = pl.pallas_call(
        _bn_body,
        out_shape=jax.ShapeDtypeStruct((B, M, N), jnp.float32),
        grid=(B // nb2,),
        in_specs=[
            pl.BlockSpec((nb2, M, N), lambda i: (i, 0, 0)),
            pl.BlockSpec((1, N), lambda i: (0, 0)),
            pl.BlockSpec((1, N), lambda i: (0, 0)),
        ],
        out_specs=pl.BlockSpec((nb2, M, N), lambda i: (i, 0, 0)),
        compiler_params=pltpu.CompilerParams(
            dimension_semantics=("parallel",)),
    )(t3, scale, shift)

    return y.reshape(B, T2, Cout, N).transpose(0, 3, 1, 2)


# speedup vs baseline: 3.8189x; 1.0022x over previous
import functools

import jax
import jax.numpy as jnp
from jax.experimental import pallas as pl
from jax.experimental.pallas import tpu as pltpu


def _ceil_to(x, m):
    return -(-x // m) * m


def _conv_as_matmul3(w3, b3, t_in, seg):
    _, K, cin, cout = w3.shape
    t_out = t_in - K + 1
    v = jnp.concatenate(
        [w3, jnp.zeros((3, t_out, cin, cout), w3.dtype)], axis=1)
    flat = jnp.broadcast_to(
        v[:, None], (3, t_out, t_in + 1, cin, cout)).reshape(
            3, t_out * (t_in + 1), cin, cout)[:, :t_out * t_in]
    P = flat.reshape(3, t_out, t_in, cin, cout)
    W = P.transpose(2, 3, 0, 1, 4).reshape(t_in * cin, 3, t_out * cout)
    W = jnp.pad(W, ((0, 0), (0, 0), (0, seg - t_out * cout)))
    bias = jnp.broadcast_to(b3[:, None, :], (3, t_out, cout)).reshape(
        3, t_out * cout)
    bias = jnp.pad(bias, ((0, 0), (0, seg - t_out * cout)))
    return W.reshape(t_in * cin, 3 * seg), bias.reshape(3 * seg, 1)


def _gated_rows(pre, seg):
    return jnp.maximum(
        pre[:seg] * jax.nn.sigmoid(pre[seg:2 * seg]) + pre[2 * seg:],
        0.0)


def _fused_body(x_ref, a_ref, w1_ref, b1_ref, th_ref, w2_ref, one_ref,
                o_ref, s1_ref, s2_ref, *, nb, n, m):
    seg1 = w1_ref.shape[0] // 3
    seg2 = w2_ref.shape[0] // 3
    w1 = w1_ref[...]
    b1 = b1_ref[...]
    th = th_ref[...]
    w2 = w2_ref[...]
    akt = a_ref[...]
    onez = one_ref[...]
    s1_acc = jnp.zeros((1, 2 * n), jnp.float32)
    s2_acc = jnp.zeros((1, 2 * n), jnp.float32)

    for p in range(nb // 2):
        xp = jnp.concatenate(
            [x_ref[2 * p], x_ref[2 * p + 1]], axis=1).astype(jnp.bfloat16)
        pre1 = jnp.dot(w1, xp, preferred_element_type=jnp.float32) + b1
        h1 = _gated_rows(pre1, seg1).astype(jnp.bfloat16)
        pj = jnp.dot(th, h1, preferred_element_type=jnp.float32
                     ).astype(jnp.bfloat16)
        mixed = jnp.dot(pj, akt, preferred_element_type=jnp.float32)
        h2 = jnp.concatenate(
            [jnp.maximum(mixed, 0.0).astype(jnp.bfloat16), onez], axis=0)
        pre2 = jnp.dot(w2, h2, preferred_element_type=jnp.float32)
        out = _gated_rows(pre2, seg2)[:m]
        o_ref[2 * p] = out[:, :n].astype(jnp.bfloat16)
        o_ref[2 * p + 1] = out[:, n:].astype(jnp.bfloat16)
        s1_acc = s1_acc + jnp.sum(out, axis=0, keepdims=True)
        s2_acc = s2_acc + jnp.sum(out * out, axis=0, keepdims=True)

    s1_ref[...] = (s1_acc[:, :n] + s1_acc[:, n:]).reshape(1, 1, n)
    s2_ref[...] = (s2_acc[:, :n] + s2_acc[:, n:]).reshape(1, 1, n)


def _bn_body(t_ref, sc_ref, sh_ref, y_ref):
    y_ref[...] = (t_ref[...].astype(jnp.float32)
                  * sc_ref[...][None, :, :] + sh_ref[...][None, :, :])


def kernel(t1_w, t1_b, theta1, t2_w, t2_b, bn_gamma, bn_beta, X, A_hat):
    B, N, T, Cin = X.shape
    K = t1_w.shape[1]
    Cout = t1_w.shape[3]
    S = theta1.shape[1]
    T1 = T - (K - 1)
    T2 = T1 - (K - 1)
    M = T2 * Cout
    seg1 = _ceil_to(T1 * Cout, 128)
    seg2 = _ceil_to(M, 128)
    pjw = _ceil_to(T1 * S + 1, 8)

    w1f, b1 = _conv_as_matmul3(t1_w, t1_b, T, seg1)
    w2f, b2 = _conv_as_matmul3(t2_w, t2_b, T1, seg2)
    w1t = w1f.T.astype(jnp.bfloat16)
    w2t = jnp.concatenate(
        [w2f, b2.reshape(1, 3 * seg2),
         jnp.zeros((pjw - T1 * S - 1, 3 * seg2), w2f.dtype)],
        axis=0).T.astype(jnp.bfloat16)
    eye = jnp.eye(T1, dtype=theta1.dtype)
    tht = (eye[:, None, :, None] * theta1.T[None, :, None, :]).reshape(
        T1 * S, T1 * Cout)
    tht = jnp.pad(tht, ((0, 0), (0, seg1 - T1 * Cout))).astype(jnp.bfloat16)
    akt = jnp.kron(jnp.eye(2, dtype=A_hat.dtype), A_hat.T
                   ).astype(jnp.bfloat16)
    row = jax.lax.broadcasted_iota(jnp.int32, (pjw - T1 * S, 1), 0)
    onez = jnp.where(row == 0, jnp.bfloat16(1.0), jnp.bfloat16(0.0))
    onez = jnp.broadcast_to(onez, (pjw - T1 * S, 2 * N))

    x_t = X.transpose(0, 2, 3, 1).reshape(B, T * Cin, N)

    nb = next(t for t in (32, 16, 8, 4, 2) if B % t == 0)
    nsteps = B // nb
    t3, s1, s2 = pl.pallas_call(
        functools.partial(_fused_body, nb=nb, n=N, m=M),
        out_shape=(jax.ShapeDtypeStruct((B, M, N), jnp.bfloat16),
                   jax.ShapeDtypeStruct((nsteps, 1, N), jnp.float32),
                   jax.ShapeDtypeStruct((nsteps, 1, N), jnp.float32)),
        grid=(nsteps,),
        in_specs=[
            pl.BlockSpec((nb, T * Cin, N), lambda i: (i, 0, 0)),
            pl.BlockSpec((2 * N, 2 * N), lambda i: (0, 0)),
            pl.BlockSpec((3 * seg1, T * Cin), lambda i: (0, 0)),
            pl.BlockSpec((3 * seg1, 1), lambda i: (0, 0)),
            pl.BlockSpec((T1 * S, seg1), lambda i: (0, 0)),
            pl.BlockSpec((3 * seg2, pjw), lambda i: (0, 0)),
            pl.BlockSpec((pjw - T1 * S, 2 * N), lambda i: (0, 0)),
        ],
        out_specs=(
            pl.BlockSpec((nb, M, N), lambda i: (i, 0, 0)),
            pl.BlockSpec((1, 1, N), lambda i: (i, 0, 0)),
            pl.BlockSpec((1, 1, N), lambda i: (i, 0, 0)),
        ),
        compiler_params=pltpu.CompilerParams(
            dimension_semantics=("parallel",)),
    )(x_t, akt, w1t, b1, tht, w2t, onez)

    cnt = float(B * M)
    mean = jnp.sum(s1[:, 0, :], axis=0) / cnt
    var = jnp.maximum(jnp.sum(s2[:, 0, :], axis=0) / cnt - mean * mean, 0.0)
    inv = jax.lax.rsqrt(var + 1e-5)
    g = bn_gamma[:, 0]
    scale = (g * inv)[None, :]
    shift = (bn_beta[:, 0] - mean * g * inv)[None, :]

    nb2 = 32 if B % 32 == 0 else nb
    y = pl.pallas_call(
        _bn_body,
        out_shape=jax.ShapeDtypeStruct((B, M, N), jnp.float32),
        grid=(B // nb2,),
        in_specs=[
            pl.BlockSpec((nb2, M, N), lambda i: (i, 0, 0)),
            pl.BlockSpec((1, N), lambda i: (0, 0)),
            pl.BlockSpec((1, N), lambda i: (0, 0)),
        ],
        out_specs=pl.BlockSpec((nb2, M, N), lambda i: (i, 0, 0)),
        compiler_params=pltpu.CompilerParams(
            dimension_semantics=("parallel",)),
    )(t3, scale, shift)

    return y.reshape(B, T2, Cout, N).transpose(0, 3, 1, 2)
